# baseline (speedup 1.0000x reference)
import jax
import jax.numpy as jnp
from jax import lax
from jax.experimental import pallas as pl
from jax.experimental.pallas import tpu as pltpu
from jax.experimental.pallas import tpu_sc as plsc

_NC = 2
_NS = 16
_BLK = 576
_MAXB = 14
_HB = 512


def _gate_route_kernel(x_ref, wg_ref, slot_ref, wk_ref, meta_ref):
    x = x_ref[...]
    wg = wg_ref[...]
    logits = lax.dot_general(x, wg, (((1,), (1,)), ((), ())),
                             preferred_element_type=jnp.float32)
    m = jnp.max(logits, axis=1, keepdims=True)
    p = jnp.exp(logits - m)
    g = p / jnp.sum(p, axis=1, keepdims=True)
    num_e = g.shape[1]
    n = x.shape[0]
    iota = lax.broadcasted_iota(jnp.int32, g.shape, 1)
    v1 = jnp.max(g, axis=1, keepdims=True)
    i1 = jnp.min(jnp.where(g >= v1, iota, num_e), axis=1, keepdims=True)
    g2 = jnp.where(iota == i1, -1.0, g)
    v2 = jnp.max(g2, axis=1, keepdims=True)
    i2 = jnp.min(jnp.where(g2 >= v2, iota, num_e), axis=1, keepdims=True)
    denom = v1 + v2 + 1e-9
    wk_ref[pl.ds(0, n), :] = v1 / denom
    wk_ref[pl.ds(n, n), :] = v2 / denom

    m1 = (iota == i1).astype(jnp.int32)
    m2 = (iota == i2).astype(jnp.int32)
    hot = jnp.concatenate([m1, m2], axis=0)
    acc = hot
    sh = 1
    while sh < 2 * n:
        top = jnp.zeros((sh, num_e), jnp.int32)
        acc = acc + jnp.concatenate([top, acc[:2 * n - sh]], axis=0)
        sh *= 2
    cs = acc
    tot = cs[2 * n - 1:2 * n, :]
    nb = (tot + (_BLK - 1)) // _BLK

    parts = []
    run = jnp.zeros((1, 1), jnp.int32)
    for e in range(num_e):
        nbe = lax.slice(nb, (0, e), (1, e + 1))
        parts.append(run)
        run = run + nbe
    excl_row = jnp.concatenate(parts, axis=1)
    pstart = excl_row * _BLK
    slot_ref[...] = jnp.sum(hot * (cs - 1 + pstart), axis=1, keepdims=True)

    iota32 = lax.broadcasted_iota(jnp.int32, (1, 32), 1)
    nbl = run
    bev = jnp.zeros((1, 32), jnp.int32)
    tail = jnp.zeros((1, 1), jnp.int32)
    run2 = jnp.zeros((1, 1), jnp.int32)
    for e in range(num_e):
        run2 = run2 + lax.slice(nb, (0, e), (1, e + 1))
        bev = bev + jnp.where(run2 <= iota32, 1, 0)
        tail = tail + jnp.where(run2 <= nbl - 1, 1, 0)
    meta_ref[...] = jnp.where(iota32 < _MAXB, jnp.minimum(bev, tail), nbl)


def _route_body(slotflat, wkflat, x, xs, wslot,
                xv, slots_v, wkv, semx, sem1, sem2):
    c = lax.axis_index("c")
    s = lax.axis_index("s")
    w = 2 * s + c
    tok_base = (w % 16) * 128
    dx = pltpu.async_copy(x.at[pl.ds(tok_base, 128)], xv, semx)
    pltpu.sync_copy(slotflat.at[pl.ds(w * 128, 128)], slots_v)
    pltpu.sync_copy(wkflat.at[pl.ds(w * 128, 128)], wkv)
    dx.wait()
    d1 = pltpu.async_copy(xv, xs.at[slots_v], sem1)
    d2 = pltpu.async_copy(wkv, wslot.at[slots_v], sem2)
    d1.wait()
    d2.wait()


def _ffn_kernel(m_ref, xs_ref, ws_ref, w1_ref, w2_ref, w3_ref, ys_ref):
    b = pl.program_id(0)
    hb = pl.program_id(1)
    nhb = pl.num_programs(1)

    @pl.when(b < m_ref[_MAXB])
    def _():
        xs = xs_ref[...]
        w1 = w1_ref[0]
        w2 = w2_ref[0]
        w3 = w3_ref[0]
        h1 = lax.dot_general(xs, w1, (((1,), (1,)), ((), ())),
                             preferred_element_type=jnp.float32)
        h2 = lax.dot_general(xs, w2, (((1,), (1,)), ((), ())),
                             preferred_element_type=jnp.float32)
        sg = 1.0 / (1.0 + jnp.exp(-h1))
        gmid = h1 * sg * h2
        part = lax.dot_general(gmid, w3, (((1,), (1,)), ((), ())),
                               preferred_element_type=jnp.float32)

        @pl.when(hb == 0)
        def _():
            ys_ref[...] = part

        @pl.when(hb != 0)
        def _():
            ys_ref[...] += part

        @pl.when(hb == nhb - 1)
        def _():
            ys_ref[...] *= ws_ref[...]


def _combine_body(ys, slotflat, out, se_v, so_v, ya, yb, sem1, sem2):
    c = lax.axis_index("c")
    s = lax.axis_index("s")
    w = 2 * s + c
    base = w * 64
    n_tok = out.shape[0]
    pltpu.sync_copy(slotflat.at[pl.ds(base, 64)], se_v)
    pltpu.sync_copy(slotflat.at[pl.ds(n_tok + base, 64)], so_v)
    g1 = pltpu.async_copy(ys.at[se_v], ya, sem1)
    g2 = pltpu.async_copy(ys.at[so_v], yb, sem2)
    g1.wait()
    g2.wait()

    nvec = ya.shape[1] // 16

    def add_row(j, _):
        for v in range(nvec):
            sl = pl.ds(v * 16, 16)
            ya[j, sl] = ya[j, sl] + yb[j, sl]
        return 0
    lax.fori_loop(0, 64, add_row, 0)
    pltpu.sync_copy(ya, out.at[pl.ds(base, 64)])


def kernel(x, Wg, W1, W2, W3):
    n_tok, d_model = x.shape
    num_e, hidden, _ = W1.shape
    nhb = hidden // _HB
    n_slot = _MAXB * _BLK

    slotflat2, wkflat2, meta2 = pl.pallas_call(
        _gate_route_kernel,
        out_shape=[
            jax.ShapeDtypeStruct((2 * n_tok, 1), jnp.int32),
            jax.ShapeDtypeStruct((2 * n_tok, 1), jnp.float32),
            jax.ShapeDtypeStruct((1, 32), jnp.int32),
        ],
    )(x, Wg)
    slotflat = slotflat2.reshape(-1)
    wkflat = wkflat2.reshape(-1)
    meta = meta2.reshape(-1)

    mesh = plsc.VectorSubcoreMesh(core_axis_name="c", subcore_axis_name="s")
    route = pl.kernel(
        _route_body,
        compiler_params=pltpu.CompilerParams(needs_layout_passes=False),
        out_type=[
            jax.ShapeDtypeStruct((n_slot, d_model), jnp.float32),
            jax.ShapeDtypeStruct((n_slot,), jnp.float32),
        ],
        mesh=mesh,
        scratch_types=[
            pltpu.VMEM((128, d_model), jnp.float32),
            pltpu.VMEM((128,), jnp.int32),
            pltpu.VMEM((128,), jnp.float32),
            pltpu.SemaphoreType.DMA,
            pltpu.SemaphoreType.DMA,
            pltpu.SemaphoreType.DMA,
        ],
    )
    xs, wslot = route(slotflat, wkflat, x)

    def _row_idx(b, hb, m):
        return (jnp.minimum(b, m[_MAXB] - 1), 0)

    def _w12_idx(b, hb, m):
        return (m[b], jnp.where(b < m[_MAXB], hb, nhb - 1), 0)

    def _w3_idx(b, hb, m):
        return (m[b], 0, jnp.where(b < m[_MAXB], hb, nhb - 1))

    ys = pl.pallas_call(
        _ffn_kernel,
        grid_spec=pltpu.PrefetchScalarGridSpec(
            num_scalar_prefetch=1,
            grid=(_MAXB, nhb),
            in_specs=[
                pl.BlockSpec((_BLK, d_model), _row_idx),
                pl.BlockSpec((_BLK, 1), _row_idx),
                pl.BlockSpec((1, _HB, d_model), _w12_idx),
                pl.BlockSpec((1, _HB, d_model), _w12_idx),
                pl.BlockSpec((1, d_model, _HB), _w3_idx),
            ],
            out_specs=pl.BlockSpec((_BLK, d_model), _row_idx),
        ),
        out_shape=jax.ShapeDtypeStruct((n_slot, d_model), jnp.float32),
    )(meta, xs, wslot.reshape(n_slot, 1), W1, W2, W3)

    combine = pl.kernel(
        _combine_body,
        out_type=jax.ShapeDtypeStruct((n_tok, d_model), jnp.float32),
        mesh=plsc.VectorSubcoreMesh(core_axis_name="c",
                                    subcore_axis_name="s"),
        scratch_types=[
            pltpu.VMEM((64,), jnp.int32),
            pltpu.VMEM((64,), jnp.int32),
            pltpu.VMEM((64, d_model), jnp.float32),
            pltpu.VMEM((64, d_model), jnp.float32),
            pltpu.SemaphoreType.DMA,
            pltpu.SemaphoreType.DMA,
        ],
    )
    out = combine(ys, slotflat)
    return out


# speedup vs baseline: 1.0641x; 1.0641x over previous
import jax
import jax.numpy as jnp
from jax import lax
from jax.experimental import pallas as pl
from jax.experimental.pallas import tpu as pltpu
from jax.experimental.pallas import tpu_sc as plsc

_NC = 2
_NS = 16
_BLK = 576
_MAXB = 14
_HB = 768


def _gate_route_kernel(x_ref, wg_ref, slot_ref, wk_ref, meta_ref):
    x = x_ref[...]
    wg = wg_ref[...]
    logits = lax.dot_general(x, wg, (((1,), (1,)), ((), ())),
                             preferred_element_type=jnp.float32)
    m = jnp.max(logits, axis=1, keepdims=True)
    p = jnp.exp(logits - m)
    g = p / jnp.sum(p, axis=1, keepdims=True)
    num_e = g.shape[1]
    n = x.shape[0]
    iota = lax.broadcasted_iota(jnp.int32, g.shape, 1)
    v1 = jnp.max(g, axis=1, keepdims=True)
    i1 = jnp.min(jnp.where(g >= v1, iota, num_e), axis=1, keepdims=True)
    g2 = jnp.where(iota == i1, -1.0, g)
    v2 = jnp.max(g2, axis=1, keepdims=True)
    i2 = jnp.min(jnp.where(g2 >= v2, iota, num_e), axis=1, keepdims=True)
    denom = v1 + v2 + 1e-9
    wk_ref[pl.ds(0, n), :] = v1 / denom
    wk_ref[pl.ds(n, n), :] = v2 / denom

    m1 = (iota == i1).astype(jnp.int32)
    m2 = (iota == i2).astype(jnp.int32)
    hot = jnp.concatenate([m1, m2], axis=0)
    acc = hot
    sh = 1
    while sh < 2 * n:
        top = jnp.zeros((sh, num_e), jnp.int32)
        acc = acc + jnp.concatenate([top, acc[:2 * n - sh]], axis=0)
        sh *= 2
    cs = acc
    tot = cs[2 * n - 1:2 * n, :]
    nb = (tot + (_BLK - 1)) // _BLK

    parts = []
    run = jnp.zeros((1, 1), jnp.int32)
    for e in range(num_e):
        nbe = lax.slice(nb, (0, e), (1, e + 1))
        parts.append(run)
        run = run + nbe
    excl_row = jnp.concatenate(parts, axis=1)
    pstart = excl_row * _BLK
    slot_ref[...] = jnp.sum(hot * (cs - 1 + pstart), axis=1, keepdims=True)

    iota32 = lax.broadcasted_iota(jnp.int32, (1, 32), 1)
    nbl = run
    bev = jnp.zeros((1, 32), jnp.int32)
    tail = jnp.zeros((1, 1), jnp.int32)
    run2 = jnp.zeros((1, 1), jnp.int32)
    for e in range(num_e):
        run2 = run2 + lax.slice(nb, (0, e), (1, e + 1))
        bev = bev + jnp.where(run2 <= iota32, 1, 0)
        tail = tail + jnp.where(run2 <= nbl - 1, 1, 0)
    meta_ref[...] = jnp.where(iota32 < _MAXB, jnp.minimum(bev, tail), nbl)


def _route_body(slotflat, wkflat, x, xs, wslot,
                xv, slots_v, wkv, semx, sem1, sem2):
    c = lax.axis_index("c")
    s = lax.axis_index("s")
    w = 2 * s + c
    tok_base = (w % 16) * 128
    dx = pltpu.async_copy(x.at[pl.ds(tok_base, 128)], xv, semx)
    pltpu.sync_copy(slotflat.at[pl.ds(w * 128, 128)], slots_v)
    pltpu.sync_copy(wkflat.at[pl.ds(w * 128, 128)], wkv)
    dx.wait()
    d1 = pltpu.async_copy(xv, xs.at[slots_v], sem1)
    d2 = pltpu.async_copy(wkv, wslot.at[slots_v], sem2)
    d1.wait()
    d2.wait()


def _ffn_kernel(m_ref, xs_ref, ws_ref, w1_ref, w2_ref, w3_ref, ys_ref):
    b = pl.program_id(0)
    hb = pl.program_id(1)
    nhb = pl.num_programs(1)

    @pl.when(b < m_ref[_MAXB])
    def _():
        xs = xs_ref[...]
        w1 = w1_ref[0]
        w2 = w2_ref[0]
        w3 = w3_ref[0]
        h1 = lax.dot_general(xs, w1, (((1,), (1,)), ((), ())),
                             preferred_element_type=jnp.float32)
        h2 = lax.dot_general(xs, w2, (((1,), (1,)), ((), ())),
                             preferred_element_type=jnp.float32)
        sg = 1.0 / (1.0 + jnp.exp(-h1))
        gmid = h1 * sg * h2
        part = lax.dot_general(gmid, w3, (((1,), (1,)), ((), ())),
                               preferred_element_type=jnp.float32)

        @pl.when(hb == 0)
        def _():
            ys_ref[...] = part

        @pl.when(hb != 0)
        def _():
            ys_ref[...] += part

        @pl.when(hb == nhb - 1)
        def _():
            ys_ref[...] *= ws_ref[...]


def _combine_body(ys, slotflat, out, se_v, so_v, ya, yb, sem1, sem2):
    c = lax.axis_index("c")
    s = lax.axis_index("s")
    w = 2 * s + c
    base = w * 64
    n_tok = out.shape[0]
    pltpu.sync_copy(slotflat.at[pl.ds(base, 64)], se_v)
    pltpu.sync_copy(slotflat.at[pl.ds(n_tok + base, 64)], so_v)
    g1 = pltpu.async_copy(ys.at[se_v], ya, sem1)
    g2 = pltpu.async_copy(ys.at[so_v], yb, sem2)
    g1.wait()
    g2.wait()

    nvec = ya.shape[1] // 16

    def add_row(j, _):
        for v in range(nvec):
            sl = pl.ds(v * 16, 16)
            ya[j, sl] = ya[j, sl] + yb[j, sl]
        return 0
    lax.fori_loop(0, 64, add_row, 0)
    pltpu.sync_copy(ya, out.at[pl.ds(base, 64)])


def kernel(x, Wg, W1, W2, W3):
    n_tok, d_model = x.shape
    num_e, hidden, _ = W1.shape
    nhb = hidden // _HB
    n_slot = _MAXB * _BLK

    slotflat2, wkflat2, meta2 = pl.pallas_call(
        _gate_route_kernel,
        out_shape=[
            jax.ShapeDtypeStruct((2 * n_tok, 1), jnp.int32),
            jax.ShapeDtypeStruct((2 * n_tok, 1), jnp.float32),
            jax.ShapeDtypeStruct((1, 32), jnp.int32),
        ],
    )(x, Wg)
    slotflat = slotflat2.reshape(-1)
    wkflat = wkflat2.reshape(-1)
    meta = meta2.reshape(-1)

    mesh = plsc.VectorSubcoreMesh(core_axis_name="c", subcore_axis_name="s")
    route = pl.kernel(
        _route_body,
        compiler_params=pltpu.CompilerParams(needs_layout_passes=False),
        out_type=[
            jax.ShapeDtypeStruct((n_slot, d_model), jnp.float32),
            jax.ShapeDtypeStruct((n_slot,), jnp.float32),
        ],
        mesh=mesh,
        scratch_types=[
            pltpu.VMEM((128, d_model), jnp.float32),
            pltpu.VMEM((128,), jnp.int32),
            pltpu.VMEM((128,), jnp.float32),
            pltpu.SemaphoreType.DMA,
            pltpu.SemaphoreType.DMA,
            pltpu.SemaphoreType.DMA,
        ],
    )
    xs, wslot = route(slotflat, wkflat, x)

    def _row_idx(b, hb, m):
        return (jnp.minimum(b, m[_MAXB] - 1), 0)

    def _w12_idx(b, hb, m):
        return (m[b], jnp.where(b < m[_MAXB], hb, nhb - 1), 0)

    def _w3_idx(b, hb, m):
        return (m[b], 0, jnp.where(b < m[_MAXB], hb, nhb - 1))

    ys = pl.pallas_call(
        _ffn_kernel,
        grid_spec=pltpu.PrefetchScalarGridSpec(
            num_scalar_prefetch=1,
            grid=(_MAXB, nhb),
            in_specs=[
                pl.BlockSpec((_BLK, d_model), _row_idx),
                pl.BlockSpec((_BLK, 1), _row_idx),
                pl.BlockSpec((1, _HB, d_model), _w12_idx),
                pl.BlockSpec((1, _HB, d_model), _w12_idx),
                pl.BlockSpec((1, d_model, _HB), _w3_idx),
            ],
            out_specs=pl.BlockSpec((_BLK, d_model), _row_idx),
        ),
        out_shape=jax.ShapeDtypeStruct((n_slot, d_model), jnp.float32),
    )(meta, xs, wslot.reshape(n_slot, 1), W1, W2, W3)

    combine = pl.kernel(
        _combine_body,
        out_type=jax.ShapeDtypeStruct((n_tok, d_model), jnp.float32),
        mesh=plsc.VectorSubcoreMesh(core_axis_name="c",
                                    subcore_axis_name="s"),
        scratch_types=[
            pltpu.VMEM((64,), jnp.int32),
            pltpu.VMEM((64,), jnp.int32),
            pltpu.VMEM((64, d_model), jnp.float32),
            pltpu.VMEM((64, d_model), jnp.float32),
            pltpu.SemaphoreType.DMA,
            pltpu.SemaphoreType.DMA,
        ],
    )
    out = combine(ys, slotflat)
    return out


# speedup vs baseline: 1.0675x; 1.0032x over previous
import jax
import jax.numpy as jnp
from jax import lax
from jax.experimental import pallas as pl
from jax.experimental.pallas import tpu as pltpu
from jax.experimental.pallas import tpu_sc as plsc

_NC = 2
_NS = 16
_BLK = 576
_MAXB = 15
_HB = 768


def _gate_route_kernel(x_ref, wg_ref, slot_ref, wk_ref, meta_ref):
    x = x_ref[...]
    wg = wg_ref[...]
    logits = lax.dot_general(x, wg, (((1,), (1,)), ((), ())),
                             preferred_element_type=jnp.float32)
    m = jnp.max(logits, axis=1, keepdims=True)
    p = jnp.exp(logits - m)
    g = p / jnp.sum(p, axis=1, keepdims=True)
    num_e = g.shape[1]
    n = x.shape[0]
    iota = lax.broadcasted_iota(jnp.int32, g.shape, 1)
    v1 = jnp.max(g, axis=1, keepdims=True)
    i1 = jnp.min(jnp.where(g >= v1, iota, num_e), axis=1, keepdims=True)
    g2 = jnp.where(iota == i1, -1.0, g)
    v2 = jnp.max(g2, axis=1, keepdims=True)
    i2 = jnp.min(jnp.where(g2 >= v2, iota, num_e), axis=1, keepdims=True)
    denom = v1 + v2 + 1e-9
    wk_ref[pl.ds(0, n), :] = v1 / denom
    wk_ref[pl.ds(n, n), :] = v2 / denom

    m1 = (iota == i1).astype(jnp.int32)
    m2 = (iota == i2).astype(jnp.int32)
    hot = jnp.concatenate([m1, m2], axis=0)
    acc = hot
    sh = 1
    while sh < 2 * n:
        top = jnp.zeros((sh, num_e), jnp.int32)
        acc = acc + jnp.concatenate([top, acc[:2 * n - sh]], axis=0)
        sh *= 2
    cs = acc
    tot = cs[2 * n - 1:2 * n, :]
    nb = (tot + (_BLK - 1)) // _BLK

    parts = []
    run = jnp.zeros((1, 1), jnp.int32)
    for e in range(num_e):
        nbe = lax.slice(nb, (0, e), (1, e + 1))
        parts.append(run)
        run = run + nbe
    excl_row = jnp.concatenate(parts, axis=1)
    pstart = excl_row * _BLK
    slot_ref[...] = jnp.sum(hot * (cs - 1 + pstart), axis=1, keepdims=True)

    iota32 = lax.broadcasted_iota(jnp.int32, (1, 32), 1)
    nbl = run
    bev = jnp.zeros((1, 32), jnp.int32)
    tail = jnp.zeros((1, 1), jnp.int32)
    run2 = jnp.zeros((1, 1), jnp.int32)
    for e in range(num_e):
        run2 = run2 + lax.slice(nb, (0, e), (1, e + 1))
        bev = bev + jnp.where(run2 <= iota32, 1, 0)
        tail = tail + jnp.where(run2 <= nbl - 1, 1, 0)
    meta_ref[...] = jnp.where(iota32 < _MAXB, jnp.minimum(bev, tail), nbl)


def _route_body(slotflat, wkflat, x, xs, wslot,
                xv, slots_v, wkv, semx, sem1, sem2):
    c = lax.axis_index("c")
    s = lax.axis_index("s")
    w = 2 * s + c
    tok_base = (w % 16) * 128
    dx = pltpu.async_copy(x.at[pl.ds(tok_base, 128)], xv, semx)
    pltpu.sync_copy(slotflat.at[pl.ds(w * 128, 128)], slots_v)
    pltpu.sync_copy(wkflat.at[pl.ds(w * 128, 128)], wkv)
    dx.wait()
    d1 = pltpu.async_copy(xv, xs.at[slots_v], sem1)
    d2 = pltpu.async_copy(wkv, wslot.at[slots_v], sem2)
    d1.wait()
    d2.wait()


def _ffn_kernel(m_ref, xs_ref, ws_ref, w1_ref, w2_ref, w3_ref, ys_ref):
    b = pl.program_id(0)
    hb = pl.program_id(1)
    nhb = pl.num_programs(1)

    @pl.when(b < m_ref[_MAXB])
    def _():
        xs = xs_ref[...]
        w1 = w1_ref[0]
        w2 = w2_ref[0]
        w3 = w3_ref[0]
        h1 = lax.dot_general(xs, w1, (((1,), (1,)), ((), ())),
                             preferred_element_type=jnp.float32)
        h2 = lax.dot_general(xs, w2, (((1,), (1,)), ((), ())),
                             preferred_element_type=jnp.float32)
        sg = 1.0 / (1.0 + jnp.exp(-h1))
        gmid = h1 * sg * h2
        part = lax.dot_general(gmid, w3, (((1,), (1,)), ((), ())),
                               preferred_element_type=jnp.float32)

        @pl.when(hb == 0)
        def _():
            ys_ref[...] = part

        @pl.when(hb != 0)
        def _():
            ys_ref[...] += part

        @pl.when(hb == nhb - 1)
        def _():
            ys_ref[...] *= ws_ref[...]


def _combine_body(ys, slotflat, out, se_v, so_v, ya, yb, sem1, sem2):
    c = lax.axis_index("c")
    s = lax.axis_index("s")
    w = 2 * s + c
    base = w * 64
    n_tok = out.shape[0]
    pltpu.sync_copy(slotflat.at[pl.ds(base, 64)], se_v)
    pltpu.sync_copy(slotflat.at[pl.ds(n_tok + base, 64)], so_v)
    g1 = pltpu.async_copy(ys.at[se_v], ya, sem1)
    g2 = pltpu.async_copy(ys.at[so_v], yb, sem2)
    g1.wait()
    g2.wait()

    nvec = ya.shape[1] // 16

    def add_row(j, _):
        for v in range(nvec):
            sl = pl.ds(v * 16, 16)
            ya[j, sl] = ya[j, sl] + yb[j, sl]
        return 0
    lax.fori_loop(0, 64, add_row, 0)
    pltpu.sync_copy(ya, out.at[pl.ds(base, 64)])


def kernel(x, Wg, W1, W2, W3):
    n_tok, d_model = x.shape
    num_e, hidden, _ = W1.shape
    nhb = hidden // _HB
    n_slot = _MAXB * _BLK

    slotflat2, wkflat2, meta2 = pl.pallas_call(
        _gate_route_kernel,
        out_shape=[
            jax.ShapeDtypeStruct((2 * n_tok, 1), jnp.int32),
            jax.ShapeDtypeStruct((2 * n_tok, 1), jnp.float32),
            jax.ShapeDtypeStruct((1, 32), jnp.int32),
        ],
    )(x, Wg)
    slotflat = slotflat2.reshape(-1)
    wkflat = wkflat2.reshape(-1)
    meta = meta2.reshape(-1)

    mesh = plsc.VectorSubcoreMesh(core_axis_name="c", subcore_axis_name="s")
    route = pl.kernel(
        _route_body,
        compiler_params=pltpu.CompilerParams(needs_layout_passes=False),
        out_type=[
            jax.ShapeDtypeStruct((n_slot, d_model), jnp.float32),
            jax.ShapeDtypeStruct((n_slot,), jnp.float32),
        ],
        mesh=mesh,
        scratch_types=[
            pltpu.VMEM((128, d_model), jnp.float32),
            pltpu.VMEM((128,), jnp.int32),
            pltpu.VMEM((128,), jnp.float32),
            pltpu.SemaphoreType.DMA,
            pltpu.SemaphoreType.DMA,
            pltpu.SemaphoreType.DMA,
        ],
    )
    xs, wslot = route(slotflat, wkflat, x)

    def _row_idx(b, hb, m):
        return (jnp.minimum(b, m[_MAXB] - 1), 0)

    def _w12_idx(b, hb, m):
        return (m[b], jnp.where(b < m[_MAXB], hb, nhb - 1), 0)

    def _w3_idx(b, hb, m):
        return (m[b], 0, jnp.where(b < m[_MAXB], hb, nhb - 1))

    ys = pl.pallas_call(
        _ffn_kernel,
        grid_spec=pltpu.PrefetchScalarGridSpec(
            num_scalar_prefetch=1,
            grid=(_MAXB, nhb),
            in_specs=[
                pl.BlockSpec((_BLK, d_model), _row_idx),
                pl.BlockSpec((_BLK, 1), _row_idx),
                pl.BlockSpec((1, _HB, d_model), _w12_idx),
                pl.BlockSpec((1, _HB, d_model), _w12_idx),
                pl.BlockSpec((1, d_model, _HB), _w3_idx),
            ],
            out_specs=pl.BlockSpec((_BLK, d_model), _row_idx),
        ),
        out_shape=jax.ShapeDtypeStruct((n_slot, d_model), jnp.float32),
    )(meta, xs, wslot.reshape(n_slot, 1), W1, W2, W3)

    combine = pl.kernel(
        _combine_body,
        out_type=jax.ShapeDtypeStruct((n_tok, d_model), jnp.float32),
        mesh=plsc.VectorSubcoreMesh(core_axis_name="c",
                                    subcore_axis_name="s"),
        scratch_types=[
            pltpu.VMEM((64,), jnp.int32),
            pltpu.VMEM((64,), jnp.int32),
            pltpu.VMEM((64, d_model), jnp.float32),
            pltpu.VMEM((64, d_model), jnp.float32),
            pltpu.SemaphoreType.DMA,
            pltpu.SemaphoreType.DMA,
        ],
    )
    out = combine(ys, slotflat)
    return out


# speedup vs baseline: 1.1108x; 1.0406x over previous
import jax
import jax.numpy as jnp
from jax import lax
from jax.experimental import pallas as pl
from jax.experimental.pallas import tpu as pltpu
from jax.experimental.pallas import tpu_sc as plsc

_NC = 2
_NS = 16
_BLK = 576
_MAXB = 15
_HB = 1024


def _gate_route_kernel(x_ref, wg_ref, slot_ref, wk_ref, meta_ref):
    x = x_ref[...]
    wg = wg_ref[...]
    logits = lax.dot_general(x, wg, (((1,), (1,)), ((), ())),
                             preferred_element_type=jnp.float32)
    m = jnp.max(logits, axis=1, keepdims=True)
    p = jnp.exp(logits - m)
    g = p / jnp.sum(p, axis=1, keepdims=True)
    num_e = g.shape[1]
    n = x.shape[0]
    iota = lax.broadcasted_iota(jnp.int32, g.shape, 1)
    v1 = jnp.max(g, axis=1, keepdims=True)
    i1 = jnp.min(jnp.where(g >= v1, iota, num_e), axis=1, keepdims=True)
    g2 = jnp.where(iota == i1, -1.0, g)
    v2 = jnp.max(g2, axis=1, keepdims=True)
    i2 = jnp.min(jnp.where(g2 >= v2, iota, num_e), axis=1, keepdims=True)
    denom = v1 + v2 + 1e-9
    wk_ref[pl.ds(0, n), :] = v1 / denom
    wk_ref[pl.ds(n, n), :] = v2 / denom

    m1 = (iota == i1).astype(jnp.int32)
    m2 = (iota == i2).astype(jnp.int32)
    hot = jnp.concatenate([m1, m2], axis=0)
    acc = hot
    sh = 1
    while sh < 2 * n:
        top = jnp.zeros((sh, num_e), jnp.int32)
        acc = acc + jnp.concatenate([top, acc[:2 * n - sh]], axis=0)
        sh *= 2
    cs = acc
    tot = cs[2 * n - 1:2 * n, :]
    nb = (tot + (_BLK - 1)) // _BLK

    parts = []
    run = jnp.zeros((1, 1), jnp.int32)
    for e in range(num_e):
        nbe = lax.slice(nb, (0, e), (1, e + 1))
        parts.append(run)
        run = run + nbe
    excl_row = jnp.concatenate(parts, axis=1)
    pstart = excl_row * _BLK
    slot_ref[...] = jnp.sum(hot * (cs - 1 + pstart), axis=1, keepdims=True)

    iota32 = lax.broadcasted_iota(jnp.int32, (1, 32), 1)
    nbl = run
    bev = jnp.zeros((1, 32), jnp.int32)
    tail = jnp.zeros((1, 1), jnp.int32)
    run2 = jnp.zeros((1, 1), jnp.int32)
    for e in range(num_e):
        run2 = run2 + lax.slice(nb, (0, e), (1, e + 1))
        bev = bev + jnp.where(run2 <= iota32, 1, 0)
        tail = tail + jnp.where(run2 <= nbl - 1, 1, 0)
    meta_ref[...] = jnp.where(iota32 < _MAXB, jnp.minimum(bev, tail), nbl)


def _route_body(slotflat, wkflat, x, xs, wslot,
                xv, slots_v, wkv, semx, sem1, sem2):
    c = lax.axis_index("c")
    s = lax.axis_index("s")
    w = 2 * s + c
    tok_base = (w % 16) * 128
    dx = pltpu.async_copy(x.at[pl.ds(tok_base, 128)], xv, semx)
    pltpu.sync_copy(slotflat.at[pl.ds(w * 128, 128)], slots_v)
    pltpu.sync_copy(wkflat.at[pl.ds(w * 128, 128)], wkv)
    dx.wait()
    d1 = pltpu.async_copy(xv, xs.at[slots_v], sem1)
    d2 = pltpu.async_copy(wkv, wslot.at[slots_v], sem2)
    d1.wait()
    d2.wait()


def _ffn_kernel(m_ref, xs_ref, ws_ref, w1_ref, w2_ref, w3_ref, ys_ref):
    b = pl.program_id(0)
    hb = pl.program_id(1)
    nhb = pl.num_programs(1)

    @pl.when(b < m_ref[_MAXB])
    def _():
        xs = xs_ref[...]
        w1 = w1_ref[0]
        w2 = w2_ref[0]
        w3 = w3_ref[0]
        h1 = lax.dot_general(xs, w1, (((1,), (1,)), ((), ())),
                             preferred_element_type=jnp.float32)
        h2 = lax.dot_general(xs, w2, (((1,), (1,)), ((), ())),
                             preferred_element_type=jnp.float32)
        sg = 1.0 / (1.0 + jnp.exp(-h1))
        gmid = h1 * sg * h2
        part = lax.dot_general(gmid, w3, (((1,), (1,)), ((), ())),
                               preferred_element_type=jnp.float32)

        @pl.when(hb == 0)
        def _():
            ys_ref[...] = part

        @pl.when(hb != 0)
        def _():
            ys_ref[...] += part

        @pl.when(hb == nhb - 1)
        def _():
            ys_ref[...] *= ws_ref[...]


def _combine_body(ys, slotflat, out, se_v, so_v, ya, yb, sem1, sem2):
    c = lax.axis_index("c")
    s = lax.axis_index("s")
    w = 2 * s + c
    base = w * 64
    n_tok = out.shape[0]
    pltpu.sync_copy(slotflat.at[pl.ds(base, 64)], se_v)
    pltpu.sync_copy(slotflat.at[pl.ds(n_tok + base, 64)], so_v)
    g1 = pltpu.async_copy(ys.at[se_v], ya, sem1)
    g2 = pltpu.async_copy(ys.at[so_v], yb, sem2)
    g1.wait()
    g2.wait()

    nvec = ya.shape[1] // 16

    def add_row(j, _):
        for v in range(nvec):
            sl = pl.ds(v * 16, 16)
            ya[j, sl] = ya[j, sl] + yb[j, sl]
        return 0
    lax.fori_loop(0, 64, add_row, 0)
    pltpu.sync_copy(ya, out.at[pl.ds(base, 64)])


def kernel(x, Wg, W1, W2, W3):
    n_tok, d_model = x.shape
    num_e, hidden, _ = W1.shape
    nhb = hidden // _HB
    n_slot = _MAXB * _BLK

    slotflat2, wkflat2, meta2 = pl.pallas_call(
        _gate_route_kernel,
        out_shape=[
            jax.ShapeDtypeStruct((2 * n_tok, 1), jnp.int32),
            jax.ShapeDtypeStruct((2 * n_tok, 1), jnp.float32),
            jax.ShapeDtypeStruct((1, 32), jnp.int32),
        ],
    )(x, Wg)
    slotflat = slotflat2.reshape(-1)
    wkflat = wkflat2.reshape(-1)
    meta = meta2.reshape(-1)

    mesh = plsc.VectorSubcoreMesh(core_axis_name="c", subcore_axis_name="s")
    route = pl.kernel(
        _route_body,
        compiler_params=pltpu.CompilerParams(needs_layout_passes=False),
        out_type=[
            jax.ShapeDtypeStruct((n_slot, d_model), jnp.float32),
            jax.ShapeDtypeStruct((n_slot,), jnp.float32),
        ],
        mesh=mesh,
        scratch_types=[
            pltpu.VMEM((128, d_model), jnp.float32),
            pltpu.VMEM((128,), jnp.int32),
            pltpu.VMEM((128,), jnp.float32),
            pltpu.SemaphoreType.DMA,
            pltpu.SemaphoreType.DMA,
            pltpu.SemaphoreType.DMA,
        ],
    )
    xs, wslot = route(slotflat, wkflat, x)

    def _row_idx(b, hb, m):
        return (jnp.minimum(b, m[_MAXB] - 1), 0)

    def _w12_idx(b, hb, m):
        return (m[b], jnp.where(b < m[_MAXB], hb, nhb - 1), 0)

    def _w3_idx(b, hb, m):
        return (m[b], 0, jnp.where(b < m[_MAXB], hb, nhb - 1))

    ys = pl.pallas_call(
        _ffn_kernel,
        grid_spec=pltpu.PrefetchScalarGridSpec(
            num_scalar_prefetch=1,
            grid=(_MAXB, nhb),
            in_specs=[
                pl.BlockSpec((_BLK, d_model), _row_idx),
                pl.BlockSpec((_BLK, 1), _row_idx),
                pl.BlockSpec((1, _HB, d_model), _w12_idx),
                pl.BlockSpec((1, _HB, d_model), _w12_idx),
                pl.BlockSpec((1, d_model, _HB), _w3_idx),
            ],
            out_specs=pl.BlockSpec((_BLK, d_model), _row_idx),
        ),
        out_shape=jax.ShapeDtypeStruct((n_slot, d_model), jnp.float32),
    )(meta, xs, wslot.reshape(n_slot, 1), W1, W2, W3)

    combine = pl.kernel(
        _combine_body,
        out_type=jax.ShapeDtypeStruct((n_tok, d_model), jnp.float32),
        mesh=plsc.VectorSubcoreMesh(core_axis_name="c",
                                    subcore_axis_name="s"),
        scratch_types=[
            pltpu.VMEM((64,), jnp.int32),
            pltpu.VMEM((64,), jnp.int32),
            pltpu.VMEM((64, d_model), jnp.float32),
            pltpu.VMEM((64, d_model), jnp.float32),
            pltpu.SemaphoreType.DMA,
            pltpu.SemaphoreType.DMA,
        ],
    )
    out = combine(ys, slotflat)
    return out


# speedup vs baseline: 1.1302x; 1.0175x over previous
import jax
import jax.numpy as jnp
from jax import lax
from jax.experimental import pallas as pl
from jax.experimental.pallas import tpu as pltpu
from jax.experimental.pallas import tpu_sc as plsc

_NC = 2
_NS = 16
_BLK = 576
_MAXB = 15
_HB = 1536


def _gate_route_kernel(x_ref, wg_ref, slot_ref, wk_ref, meta_ref):
    x = x_ref[...]
    wg = wg_ref[...]
    logits = lax.dot_general(x, wg, (((1,), (1,)), ((), ())),
                             preferred_element_type=jnp.float32)
    m = jnp.max(logits, axis=1, keepdims=True)
    p = jnp.exp(logits - m)
    g = p / jnp.sum(p, axis=1, keepdims=True)
    num_e = g.shape[1]
    n = x.shape[0]
    iota = lax.broadcasted_iota(jnp.int32, g.shape, 1)
    v1 = jnp.max(g, axis=1, keepdims=True)
    i1 = jnp.min(jnp.where(g >= v1, iota, num_e), axis=1, keepdims=True)
    g2 = jnp.where(iota == i1, -1.0, g)
    v2 = jnp.max(g2, axis=1, keepdims=True)
    i2 = jnp.min(jnp.where(g2 >= v2, iota, num_e), axis=1, keepdims=True)
    denom = v1 + v2 + 1e-9
    wk_ref[pl.ds(0, n), :] = v1 / denom
    wk_ref[pl.ds(n, n), :] = v2 / denom

    m1 = (iota == i1).astype(jnp.int32)
    m2 = (iota == i2).astype(jnp.int32)
    hot = jnp.concatenate([m1, m2], axis=0)
    acc = hot
    sh = 1
    while sh < 2 * n:
        top = jnp.zeros((sh, num_e), jnp.int32)
        acc = acc + jnp.concatenate([top, acc[:2 * n - sh]], axis=0)
        sh *= 2
    cs = acc
    tot = cs[2 * n - 1:2 * n, :]
    nb = (tot + (_BLK - 1)) // _BLK

    parts = []
    run = jnp.zeros((1, 1), jnp.int32)
    for e in range(num_e):
        nbe = lax.slice(nb, (0, e), (1, e + 1))
        parts.append(run)
        run = run + nbe
    excl_row = jnp.concatenate(parts, axis=1)
    pstart = excl_row * _BLK
    slot_ref[...] = jnp.sum(hot * (cs - 1 + pstart), axis=1, keepdims=True)

    iota32 = lax.broadcasted_iota(jnp.int32, (1, 32), 1)
    nbl = run
    bev = jnp.zeros((1, 32), jnp.int32)
    tail = jnp.zeros((1, 1), jnp.int32)
    run2 = jnp.zeros((1, 1), jnp.int32)
    for e in range(num_e):
        run2 = run2 + lax.slice(nb, (0, e), (1, e + 1))
        bev = bev + jnp.where(run2 <= iota32, 1, 0)
        tail = tail + jnp.where(run2 <= nbl - 1, 1, 0)
    meta_ref[...] = jnp.where(iota32 < _MAXB, jnp.minimum(bev, tail), nbl)


def _route_body(slotflat, wkflat, x, xs, wslot,
                xv, slots_v, wkv, semx, sem1, sem2):
    c = lax.axis_index("c")
    s = lax.axis_index("s")
    w = 2 * s + c
    tok_base = (w % 16) * 128
    dx = pltpu.async_copy(x.at[pl.ds(tok_base, 128)], xv, semx)
    pltpu.sync_copy(slotflat.at[pl.ds(w * 128, 128)], slots_v)
    pltpu.sync_copy(wkflat.at[pl.ds(w * 128, 128)], wkv)
    dx.wait()
    d1 = pltpu.async_copy(xv, xs.at[slots_v], sem1)
    d2 = pltpu.async_copy(wkv, wslot.at[slots_v], sem2)
    d1.wait()
    d2.wait()


def _ffn_kernel(m_ref, xs_ref, ws_ref, w1_ref, w2_ref, w3_ref, ys_ref):
    b = pl.program_id(0)
    hb = pl.program_id(1)
    nhb = pl.num_programs(1)

    @pl.when(b < m_ref[_MAXB])
    def _():
        xs = xs_ref[...]
        w1 = w1_ref[0]
        w2 = w2_ref[0]
        w3 = w3_ref[0]
        h1 = lax.dot_general(xs, w1, (((1,), (1,)), ((), ())),
                             preferred_element_type=jnp.float32)
        h2 = lax.dot_general(xs, w2, (((1,), (1,)), ((), ())),
                             preferred_element_type=jnp.float32)
        sg = 1.0 / (1.0 + jnp.exp(-h1))
        gmid = h1 * sg * h2
        part = lax.dot_general(gmid, w3, (((1,), (1,)), ((), ())),
                               preferred_element_type=jnp.float32)

        @pl.when(hb == 0)
        def _():
            ys_ref[...] = part

        @pl.when(hb != 0)
        def _():
            ys_ref[...] += part

        @pl.when(hb == nhb - 1)
        def _():
            ys_ref[...] *= ws_ref[...]


def _combine_body(ys, slotflat, out, se_v, so_v, ya, yb, sem1, sem2):
    c = lax.axis_index("c")
    s = lax.axis_index("s")
    w = 2 * s + c
    base = w * 64
    n_tok = out.shape[0]
    pltpu.sync_copy(slotflat.at[pl.ds(base, 64)], se_v)
    pltpu.sync_copy(slotflat.at[pl.ds(n_tok + base, 64)], so_v)
    g1 = pltpu.async_copy(ys.at[se_v], ya, sem1)
    g2 = pltpu.async_copy(ys.at[so_v], yb, sem2)
    g1.wait()
    g2.wait()

    nvec = ya.shape[1] // 16

    def add_row(j, _):
        for v in range(nvec):
            sl = pl.ds(v * 16, 16)
            ya[j, sl] = ya[j, sl] + yb[j, sl]
        return 0
    lax.fori_loop(0, 64, add_row, 0)
    pltpu.sync_copy(ya, out.at[pl.ds(base, 64)])


def kernel(x, Wg, W1, W2, W3):
    n_tok, d_model = x.shape
    num_e, hidden, _ = W1.shape
    nhb = hidden // _HB
    n_slot = _MAXB * _BLK

    slotflat2, wkflat2, meta2 = pl.pallas_call(
        _gate_route_kernel,
        out_shape=[
            jax.ShapeDtypeStruct((2 * n_tok, 1), jnp.int32),
            jax.ShapeDtypeStruct((2 * n_tok, 1), jnp.float32),
            jax.ShapeDtypeStruct((1, 32), jnp.int32),
        ],
    )(x, Wg)
    slotflat = slotflat2.reshape(-1)
    wkflat = wkflat2.reshape(-1)
    meta = meta2.reshape(-1)

    mesh = plsc.VectorSubcoreMesh(core_axis_name="c", subcore_axis_name="s")
    route = pl.kernel(
        _route_body,
        compiler_params=pltpu.CompilerParams(needs_layout_passes=False),
        out_type=[
            jax.ShapeDtypeStruct((n_slot, d_model), jnp.float32),
            jax.ShapeDtypeStruct((n_slot,), jnp.float32),
        ],
        mesh=mesh,
        scratch_types=[
            pltpu.VMEM((128, d_model), jnp.float32),
            pltpu.VMEM((128,), jnp.int32),
            pltpu.VMEM((128,), jnp.float32),
            pltpu.SemaphoreType.DMA,
            pltpu.SemaphoreType.DMA,
            pltpu.SemaphoreType.DMA,
        ],
    )
    xs, wslot = route(slotflat, wkflat, x)

    def _row_idx(b, hb, m):
        return (jnp.minimum(b, m[_MAXB] - 1), 0)

    def _w12_idx(b, hb, m):
        return (m[b], jnp.where(b < m[_MAXB], hb, nhb - 1), 0)

    def _w3_idx(b, hb, m):
        return (m[b], 0, jnp.where(b < m[_MAXB], hb, nhb - 1))

    ys = pl.pallas_call(
        _ffn_kernel,
        grid_spec=pltpu.PrefetchScalarGridSpec(
            num_scalar_prefetch=1,
            grid=(_MAXB, nhb),
            in_specs=[
                pl.BlockSpec((_BLK, d_model), _row_idx),
                pl.BlockSpec((_BLK, 1), _row_idx),
                pl.BlockSpec((1, _HB, d_model), _w12_idx),
                pl.BlockSpec((1, _HB, d_model), _w12_idx),
                pl.BlockSpec((1, d_model, _HB), _w3_idx),
            ],
            out_specs=pl.BlockSpec((_BLK, d_model), _row_idx),
        ),
        out_shape=jax.ShapeDtypeStruct((n_slot, d_model), jnp.float32),
    )(meta, xs, wslot.reshape(n_slot, 1), W1, W2, W3)

    combine = pl.kernel(
        _combine_body,
        out_type=jax.ShapeDtypeStruct((n_tok, d_model), jnp.float32),
        mesh=plsc.VectorSubcoreMesh(core_axis_name="c",
                                    subcore_axis_name="s"),
        scratch_types=[
            pltpu.VMEM((64,), jnp.int32),
            pltpu.VMEM((64,), jnp.int32),
            pltpu.VMEM((64, d_model), jnp.float32),
            pltpu.VMEM((64, d_model), jnp.float32),
            pltpu.SemaphoreType.DMA,
            pltpu.SemaphoreType.DMA,
        ],
    )
    out = combine(ys, slotflat)
    return out
